# Initial kernel scaffold; baseline (speedup 1.0000x reference)
import jax
import jax.numpy as jnp
from jax.experimental import pallas as pl


def kernel(loc_preds, conf_preds, loc_targets, conf_targets):
    raise NotImplementedError("write your pallas kernel here")


# speedup vs baseline: 1.3337x; 1.3337x over previous
import jax
import jax.numpy as jnp
from jax.experimental import pallas as pl
from jax.experimental.pallas import tpu as pltpu

_B, _P, _C = 32, 8732, 81
_PBLK = 2184
_NJ = 4
_GRID = _B * _NJ
_NEG_POS_RATIO = 3


def _phase_a(lab_ref, conf_ref, lp_ref, lt_ref, ce_ref, scal_ref):
    g = pl.program_id(0)
    j = g // _B

    @pl.when(g == 0)
    def _init():
        scal_ref[0] = 0.0
        scal_ref[1] = 0.0

    labels = lab_ref[0]
    conf = conf_ref[0]
    pid = jax.lax.broadcasted_iota(jnp.int32, (_PBLK, 1), 0) + j * _PBLK
    valid = pid < _P
    pos = labels > 0

    cls_id = jax.lax.broadcasted_iota(jnp.int32, (_PBLK, _C), 1)
    x_lab = jnp.sum(jnp.where(cls_id == labels, conf, 0.0), axis=1,
                    keepdims=True)
    m = jnp.max(conf, axis=1, keepdims=True)
    s = jnp.sum(jnp.exp(conf - m), axis=1, keepdims=True)
    ce = jnp.log(s) + m - x_lab
    ce_ref[0] = jnp.where(valid & jnp.logical_not(pos), ce, 0.0)

    d = lp_ref[0] - lt_ref[0]
    ad = jnp.abs(d)
    sl1 = jnp.where(ad < 1.0, 0.5 * d * d, ad - 0.5)
    row_l = jnp.sum(sl1, axis=1, keepdims=True)
    row_l = jnp.where(valid & pos, row_l, 0.0)
    scal_ref[0] += jnp.sum(row_l)
    scal_ref[1] += jnp.sum((valid & pos).astype(jnp.float32))


def _phase_b(ce_ref, scal_ref, out_ref):
    np_f = scal_ref[1]
    np_i = np_f.astype(jnp.int32)
    k = jnp.minimum(_NEG_POS_RATIO * np_i, _P)

    x = ce_ref[...]
    bits = jax.lax.bitcast_convert_type(x, jnp.int32)

    def group4(v):
        return (v[0:_B] + v[_B:2 * _B]
                + v[2 * _B:3 * _B] + v[3 * _B:4 * _B])

    def rep4(v):
        return jnp.concatenate([v, v, v, v], axis=0)

    def body(_, lohi):
        lo, hi = lohi
        mid = lo + ((hi - lo + 1) >> 1)
        cnt = jnp.sum((bits >= rep4(mid)).astype(jnp.int32), axis=1,
                      keepdims=True)
        take = group4(cnt) >= k
        return (jnp.where(take, mid, lo),
                jnp.where(take, hi, mid - 1))

    lo0 = jnp.zeros((_B, 1), jnp.int32)
    hi0 = jnp.full((_B, 1), 0x7f7fffff, jnp.int32)
    iters = jnp.where(_NEG_POS_RATIO * np_i >= _P, 0, 31)
    tb, _hi = jax.lax.fori_loop(0, iters, body, (lo0, hi0))
    tf = jax.lax.bitcast_convert_type(tb, jnp.float32)

    gt = bits > rep4(tb)
    sum_gt = jnp.sum(jnp.where(gt, x, 0.0), axis=1, keepdims=True)
    cnt_gt = jnp.sum(gt.astype(jnp.int32), axis=1, keepdims=True)
    row_conf = (group4(sum_gt)
                + tf * (k - group4(cnt_gt)).astype(jnp.float32))
    conf_sum = jnp.sum(row_conf)
    out_ref[0, 0] = (scal_ref[0] + conf_sum) / np_f


def _specs_a():
    return dict(
        grid=(_GRID,),
        in_specs=[
            pl.BlockSpec((1, _PBLK, 1), lambda g: (g, 0, 0)),
            pl.BlockSpec((1, _PBLK, _C), lambda g: (g % _B, g // _B, 0)),
            pl.BlockSpec((1, _PBLK, 4), lambda g: (g % _B, g // _B, 0)),
            pl.BlockSpec((1, _PBLK, 4), lambda g: (g % _B, g // _B, 0)),
        ],
        out_specs=[
            pl.BlockSpec((1, _PBLK, 1), lambda g: (g, 0, 0)),
            pl.BlockSpec(memory_space=pltpu.SMEM),
        ],
        out_shape=[
            jax.ShapeDtypeStruct((_GRID, _PBLK, 1), jnp.float32),
            jax.ShapeDtypeStruct((2,), jnp.float32),
        ],
    )


def _specs_b():
    return dict(
        in_specs=[
            pl.BlockSpec(memory_space=pltpu.VMEM),
            pl.BlockSpec(memory_space=pltpu.SMEM),
        ],
        out_specs=pl.BlockSpec(memory_space=pltpu.SMEM),
        out_shape=jax.ShapeDtypeStruct((1, 1), jnp.float32),
    )


def _prep_labels(conf_targets):
    lab = conf_targets.astype(jnp.int32)
    lab = jnp.pad(lab, ((0, 0), (0, _NJ * _PBLK - _P)))
    return lab.reshape(_B, _NJ, _PBLK).transpose(1, 0, 2).reshape(
        _GRID, _PBLK, 1)


def kernel(loc_preds, conf_preds, loc_targets, conf_targets):
    lab = _prep_labels(conf_targets)
    ce, scal = pl.pallas_call(_phase_a, **_specs_a())(
        lab, conf_preds, loc_preds, loc_targets)
    out = pl.pallas_call(_phase_b, **_specs_b())(
        ce.reshape(_GRID, _PBLK), scal)
    return out[0, 0]
